# baseline (speedup 1.0000x reference)
import functools
import jax
import jax.numpy as jnp
from jax import lax
from jax.experimental import pallas as pl
from jax.experimental.pallas import tpu as pltpu
from jax.experimental.pallas import tpu_sc as plsc

B, C, N, K = 4, 64, 2048, 20
OUT = 128
RT = 512
NC, NS = 2, 16
NW = NC * NS
LANES = 16
SB = 4


def _knn_body(x_ref, xt_ref, w_ref, pt_ref, qt_ref, idx_ref):
    xf = x_ref[0]
    xs = xt_ref[0]
    inner = lax.dot_general(xs, xf, (((0,), (0,)), ((), ())),
                            preferred_element_type=jnp.float32)
    sq = jnp.sum(xf * xf, axis=0, keepdims=True)
    sq_col = jnp.transpose(jnp.sum(xs * xs, axis=0, keepdims=True))
    nd = (2.0 * inner - sq_col) - sq

    iota_f = lax.broadcasted_iota(jnp.int32, (RT, N), 1).astype(jnp.float32)
    neg = jnp.float32(-jnp.inf)
    big = jnp.float32(N)
    cols = []
    for _ in range(K):
        m = jnp.max(nd, axis=1, keepdims=True)
        jf = jnp.min(jnp.where(nd == m, iota_f, big), axis=1, keepdims=True)
        cols.append(jf)
        nd = jnp.where(iota_f == jf, neg, nd)
    idx_ref[...] = jnp.concatenate(cols, axis=1).astype(jnp.int32)

    w1 = w_ref[:, :C]
    w21 = w_ref[:, C:] - w1
    pt_ref[...] = lax.dot_general(xs, w1, (((0,), (1,)), ((), ())),
                                  preferred_element_type=jnp.float32)
    qt_ref[...] = lax.dot_general(xs, w21, (((0,), (1,)), ((), ())),
                                  preferred_element_type=jnp.float32)


def _make_knn_call(b):
    return pl.pallas_call(
        _knn_body,
        grid=(N // RT,),
        in_specs=[
            pl.BlockSpec((1, C, N), lambda i, _b=b: (_b, 0, 0)),
            pl.BlockSpec((1, C, RT), lambda i, _b=b: (_b, 0, i)),
            pl.BlockSpec((OUT, 2 * C), lambda i: (0, 0)),
        ],
        out_specs=[
            pl.BlockSpec((RT, OUT), lambda i: (i, 0)),
            pl.BlockSpec((RT, OUT), lambda i: (i, 0)),
            pl.BlockSpec((RT, K), lambda i: (i, 0)),
        ],
        out_shape=[
            jax.ShapeDtypeStruct((N, OUT), jnp.float32),
            jax.ShapeDtypeStruct((N, OUT), jnp.float32),
            jax.ShapeDtypeStruct((N, K), jnp.int32),
        ],
    )


_knn_calls = [_make_knn_call(b) for b in range(B)]

SEG_PER_W = N // NW


@functools.partial(
    pl.kernel,
    mesh=plsc.VectorSubcoreMesh(core_axis_name="c", subcore_axis_name="s"),
    out_type=[
        jax.ShapeDtypeStruct((N, OUT), jnp.float32),
        jax.ShapeDtypeStruct((NW * 3 * OUT,), jnp.float32),
    ],
    scratch_types=[
        pltpu.VMEM((SEG_PER_W * K,), jnp.int32),
        pltpu.VMEM((SEG_PER_W, OUT), jnp.float32),
        pltpu.VMEM((SEG_PER_W, OUT), jnp.float32),
        pltpu.VMEM((SB * K, OUT), jnp.float32),
        pltpu.VMEM((SB * K, OUT), jnp.float32),
        pltpu.VMEM((SB * K, OUT), jnp.float32),
        pltpu.VMEM((SB * K, OUT), jnp.float32),
        pltpu.VMEM((3 * OUT,), jnp.float32),
        pltpu.SemaphoreType.DMA,
        pltpu.SemaphoreType.DMA,
        pltpu.SemaphoreType.DMA,
        pltpu.SemaphoreType.DMA,
    ],
)
def _sc_gather(pt_hbm, idx_hbm, q_hbm, m_hbm, part_hbm,
               idx_v, q_v, m_v, rows0_v, rows1_v, rows2_v, rows3_v, acc_v,
               sem0, sem1, sem2, sem3):
    wid = lax.axis_index("s") * NC + lax.axis_index("c")
    base = wid * SEG_PER_W
    nb = SEG_PER_W // SB
    pltpu.sync_copy(idx_hbm.at[pl.ds(base * K, SEG_PER_W * K)], idx_v)
    pltpu.sync_copy(q_hbm.at[pl.ds(base, SEG_PER_W)], q_v)
    zero = jnp.zeros((LANES,), jnp.float32)
    for c in range(3 * OUT // LANES):
        acc_v[pl.ds(c * LANES, LANES)] = zero

    def fire(g, rows_v, sem):
        return pltpu.async_copy(
            pt_hbm.at[idx_v.at[pl.ds(g * SB * K, SB * K)]], rows_v, sem)

    def drain(g, rows_v, sem):
        pltpu.make_async_copy(
            pt_hbm.at[idx_v.at[pl.ds(g * SB * K, SB * K)]], rows_v, sem
        ).wait()

    def process(g, rows_v):
        def chunk_body(c, carry):
            sl = pl.ds(c * LANES, LANES)
            for s in range(SB):
                seg = g * SB + s
                v0 = rows_v[s * K, sl]
                mx = v0
                sm = v0
                sq = v0 * v0
                for j in range(1, K):
                    v = rows_v[s * K + j, sl]
                    mx = jnp.maximum(mx, v)
                    sm = sm + v
                    sq = sq + v * v
                m_v[seg, sl] = mx
                qv = q_v[seg, sl]
                a0 = pl.ds(c * LANES, LANES)
                a1 = pl.ds(OUT + c * LANES, LANES)
                a2 = pl.ds(2 * OUT + c * LANES, LANES)
                acc_v[a0] = acc_v[a0] + sm
                acc_v[a1] = acc_v[a1] + sq
                acc_v[a2] = acc_v[a2] + qv * sm
            return carry

        lax.fori_loop(0, OUT // LANES, chunk_body, 0)

    bufs = [(rows0_v, sem0), (rows1_v, sem1), (rows2_v, sem2), (rows3_v, sem3)]
    for r in range(3):
        fire(r, bufs[r][0], bufs[r][1])

    def body(h, carry):
        for r in range(4):
            g = 4 * h + r
            rv, sm = bufs[r]
            nrv, nsm = bufs[(r + 3) % 4]

            @pl.when(g + 3 < nb)
            def _():
                fire(g + 3, nrv, nsm)
            drain(g, rv, sm)
            process(g, rv)
        return carry

    lax.fori_loop(0, nb // 4, body, 0)
    pltpu.sync_copy(m_v, m_hbm.at[pl.ds(base, SEG_PER_W)])
    pltpu.sync_copy(acc_v, part_hbm.at[pl.ds(wid * 3 * OUT, 3 * OUT)])


def _fin_body(m_ref, q_ref, p_ref, out_ref):
    m = m_ref[...]
    q = q_ref[...]
    p = p_ref[...]
    tsum = jnp.sum(p[:, 0, :], axis=0, keepdims=True)
    psq = jnp.sum(p[:, 1, :], axis=0, keepdims=True)
    cross = jnp.sum(p[:, 2, :], axis=0, keepdims=True)
    qs = jnp.sum(q, axis=0, keepdims=True)
    qs2 = jnp.sum(q * q, axis=0, keepdims=True)
    cnt = jnp.float32(N * K)
    mean = (tsum + K * qs) / cnt
    var = (psq + 2.0 * cross + K * qs2) / cnt - mean * mean
    inv = 1.0 / jnp.sqrt(var + 1e-5)
    z = (m + q - mean) * inv
    out_ref[0] = jnp.where(z >= 0, z, 0.2 * z)


_fin_call = pl.pallas_call(
    _fin_body,
    in_specs=[
        pl.BlockSpec((N, OUT), lambda: (0, 0)),
        pl.BlockSpec((N, OUT), lambda: (0, 0)),
        pl.BlockSpec((NW, 3, OUT), lambda: (0, 0, 0)),
    ],
    out_specs=pl.BlockSpec((1, N, OUT), lambda: (0, 0, 0)),
    out_shape=jax.ShapeDtypeStruct((1, N, OUT), jnp.float32),
)


def kernel(cloud, W):
    knn = [_knn_calls[b](cloud, cloud, W) for b in range(B)]
    gath = [_sc_gather(pt, idx.reshape(N * K), qt) for pt, qt, idx in knn]
    outs = [_fin_call(m, qt, parts_flat.reshape(NW, 3, OUT))
            for (pt, qt, idx), (m, parts_flat) in zip(knn, gath)]
    out = jnp.concatenate(outs, axis=0)
    return jnp.transpose(out, (0, 2, 1))


# speedup vs baseline: 1.0771x; 1.0771x over previous
import functools
import jax
import jax.numpy as jnp
from jax import lax
from jax.experimental import pallas as pl
from jax.experimental.pallas import tpu as pltpu
from jax.experimental.pallas import tpu_sc as plsc

B, C, N, K = 4, 64, 2048, 20
OUT = 128
RT = 128
NC, NS = 2, 16
NW = NC * NS
LANES = 16
SB = 4


def _knn_body(x_ref, xt_ref, w_ref, pt_ref, qt_ref, idx_ref):
    xf = x_ref[0]
    xs = xt_ref[0]
    inner = lax.dot_general(xs, xf, (((0,), (0,)), ((), ())),
                            preferred_element_type=jnp.float32)
    sq = jnp.sum(xf * xf, axis=0, keepdims=True)
    sq_col = jnp.transpose(jnp.sum(xs * xs, axis=0, keepdims=True))
    nd = (2.0 * inner - sq_col) - sq

    iota_f = lax.broadcasted_iota(jnp.int32, (RT, N), 1).astype(jnp.float32)
    neg = jnp.float32(-jnp.inf)
    big = jnp.float32(N)
    cols = []
    for _ in range(K):
        m = jnp.max(nd, axis=1, keepdims=True)
        jf = jnp.min(jnp.where(nd == m, iota_f, big), axis=1, keepdims=True)
        cols.append(jf)
        nd = jnp.where(iota_f == jf, neg, nd)
    idx_ref[...] = jnp.concatenate(cols, axis=1).astype(jnp.int32)

    w1 = w_ref[:, :C]
    w21 = w_ref[:, C:] - w1
    pt_ref[...] = lax.dot_general(xs, w1, (((0,), (1,)), ((), ())),
                                  preferred_element_type=jnp.float32)
    qt_ref[...] = lax.dot_general(xs, w21, (((0,), (1,)), ((), ())),
                                  preferred_element_type=jnp.float32)


def _make_knn_call(b):
    return pl.pallas_call(
        _knn_body,
        grid=(N // RT,),
        in_specs=[
            pl.BlockSpec((1, C, N), lambda i, _b=b: (_b, 0, 0)),
            pl.BlockSpec((1, C, RT), lambda i, _b=b: (_b, 0, i)),
            pl.BlockSpec((OUT, 2 * C), lambda i: (0, 0)),
        ],
        out_specs=[
            pl.BlockSpec((RT, OUT), lambda i: (i, 0)),
            pl.BlockSpec((RT, OUT), lambda i: (i, 0)),
            pl.BlockSpec((RT, K), lambda i: (i, 0)),
        ],
        out_shape=[
            jax.ShapeDtypeStruct((N, OUT), jnp.float32),
            jax.ShapeDtypeStruct((N, OUT), jnp.float32),
            jax.ShapeDtypeStruct((N, K), jnp.int32),
        ],
    )


_knn_calls = [_make_knn_call(b) for b in range(B)]

SEG_PER_W = N // NW


@functools.partial(
    pl.kernel,
    mesh=plsc.VectorSubcoreMesh(core_axis_name="c", subcore_axis_name="s"),
    out_type=[
        jax.ShapeDtypeStruct((N, OUT), jnp.float32),
        jax.ShapeDtypeStruct((NW * 3 * OUT,), jnp.float32),
    ],
    scratch_types=[
        pltpu.VMEM((SEG_PER_W * K,), jnp.int32),
        pltpu.VMEM((SEG_PER_W, OUT), jnp.float32),
        pltpu.VMEM((SEG_PER_W, OUT), jnp.float32),
        pltpu.VMEM((SB * K, OUT), jnp.float32),
        pltpu.VMEM((SB * K, OUT), jnp.float32),
        pltpu.VMEM((SB * K, OUT), jnp.float32),
        pltpu.VMEM((SB * K, OUT), jnp.float32),
        pltpu.VMEM((3 * OUT,), jnp.float32),
        pltpu.SemaphoreType.DMA,
        pltpu.SemaphoreType.DMA,
        pltpu.SemaphoreType.DMA,
        pltpu.SemaphoreType.DMA,
    ],
)
def _sc_gather(pt_hbm, idx_hbm, q_hbm, m_hbm, part_hbm,
               idx_v, q_v, m_v, rows0_v, rows1_v, rows2_v, rows3_v, acc_v,
               sem0, sem1, sem2, sem3):
    wid = lax.axis_index("s") * NC + lax.axis_index("c")
    base = wid * SEG_PER_W
    nb = SEG_PER_W // SB
    pltpu.sync_copy(idx_hbm.at[pl.ds(base * K, SEG_PER_W * K)], idx_v)
    pltpu.sync_copy(q_hbm.at[pl.ds(base, SEG_PER_W)], q_v)
    zero = jnp.zeros((LANES,), jnp.float32)
    for c in range(3 * OUT // LANES):
        acc_v[pl.ds(c * LANES, LANES)] = zero

    def fire(g, rows_v, sem):
        return pltpu.async_copy(
            pt_hbm.at[idx_v.at[pl.ds(g * SB * K, SB * K)]], rows_v, sem)

    def drain(g, rows_v, sem):
        pltpu.make_async_copy(
            pt_hbm.at[idx_v.at[pl.ds(g * SB * K, SB * K)]], rows_v, sem
        ).wait()

    def process(g, rows_v):
        def chunk_body(c, carry):
            sl = pl.ds(c * LANES, LANES)
            for s in range(SB):
                seg = g * SB + s
                v0 = rows_v[s * K, sl]
                mx = v0
                sm = v0
                sq = v0 * v0
                for j in range(1, K):
                    v = rows_v[s * K + j, sl]
                    mx = jnp.maximum(mx, v)
                    sm = sm + v
                    sq = sq + v * v
                m_v[seg, sl] = mx
                qv = q_v[seg, sl]
                a0 = pl.ds(c * LANES, LANES)
                a1 = pl.ds(OUT + c * LANES, LANES)
                a2 = pl.ds(2 * OUT + c * LANES, LANES)
                acc_v[a0] = acc_v[a0] + sm
                acc_v[a1] = acc_v[a1] + sq
                acc_v[a2] = acc_v[a2] + qv * sm
            return carry

        lax.fori_loop(0, OUT // LANES, chunk_body, 0)

    bufs = [(rows0_v, sem0), (rows1_v, sem1), (rows2_v, sem2), (rows3_v, sem3)]
    for r in range(3):
        fire(r, bufs[r][0], bufs[r][1])

    def body(h, carry):
        for r in range(4):
            g = 4 * h + r
            rv, sm = bufs[r]
            nrv, nsm = bufs[(r + 3) % 4]

            @pl.when(g + 3 < nb)
            def _():
                fire(g + 3, nrv, nsm)
            drain(g, rv, sm)
            process(g, rv)
        return carry

    lax.fori_loop(0, nb // 4, body, 0)
    pltpu.sync_copy(m_v, m_hbm.at[pl.ds(base, SEG_PER_W)])
    pltpu.sync_copy(acc_v, part_hbm.at[pl.ds(wid * 3 * OUT, 3 * OUT)])


def _fin_body(m_ref, q_ref, p_ref, out_ref):
    m = m_ref[...]
    q = q_ref[...]
    p = p_ref[...]
    tsum = jnp.sum(p[:, 0, :], axis=0, keepdims=True)
    psq = jnp.sum(p[:, 1, :], axis=0, keepdims=True)
    cross = jnp.sum(p[:, 2, :], axis=0, keepdims=True)
    qs = jnp.sum(q, axis=0, keepdims=True)
    qs2 = jnp.sum(q * q, axis=0, keepdims=True)
    cnt = jnp.float32(N * K)
    mean = (tsum + K * qs) / cnt
    var = (psq + 2.0 * cross + K * qs2) / cnt - mean * mean
    inv = 1.0 / jnp.sqrt(var + 1e-5)
    z = (m + q - mean) * inv
    out_ref[0] = jnp.where(z >= 0, z, 0.2 * z)


_fin_call = pl.pallas_call(
    _fin_body,
    in_specs=[
        pl.BlockSpec((N, OUT), lambda: (0, 0)),
        pl.BlockSpec((N, OUT), lambda: (0, 0)),
        pl.BlockSpec((NW, 3, OUT), lambda: (0, 0, 0)),
    ],
    out_specs=pl.BlockSpec((1, N, OUT), lambda: (0, 0, 0)),
    out_shape=jax.ShapeDtypeStruct((1, N, OUT), jnp.float32),
)


def kernel(cloud, W):
    knn = [_knn_calls[b](cloud, cloud, W) for b in range(B)]
    gath = [_sc_gather(pt, idx.reshape(N * K), qt) for pt, qt, idx in knn]
    outs = [_fin_call(m, qt, parts_flat.reshape(NW, 3, OUT))
            for (pt, qt, idx), (m, parts_flat) in zip(knn, gath)]
    out = jnp.concatenate(outs, axis=0)
    return jnp.transpose(out, (0, 2, 1))


# speedup vs baseline: 1.1111x; 1.0315x over previous
import functools
import jax
import jax.numpy as jnp
from jax import lax
from jax.experimental import pallas as pl
from jax.experimental.pallas import tpu as pltpu
from jax.experimental.pallas import tpu_sc as plsc

B, C, N, K = 4, 64, 2048, 20
OUT = 128
RT = 256
NC, NS = 2, 16
NW = NC * NS
LANES = 16
SB = 4


def _knn_body(x_ref, xt_ref, w_ref, pt_ref, qt_ref, idx_ref):
    xf = x_ref[0]
    xs = xt_ref[0]
    inner = lax.dot_general(xs, xf, (((0,), (0,)), ((), ())),
                            preferred_element_type=jnp.float32)
    sq = jnp.sum(xf * xf, axis=0, keepdims=True)
    sq_col = jnp.transpose(jnp.sum(xs * xs, axis=0, keepdims=True))
    nd = (2.0 * inner - sq_col) - sq

    iota_f = lax.broadcasted_iota(jnp.int32, (RT, N), 1).astype(jnp.float32)
    neg = jnp.float32(-jnp.inf)
    big = jnp.float32(N)
    cols = []
    for _ in range(K):
        m = jnp.max(nd, axis=1, keepdims=True)
        jf = jnp.min(jnp.where(nd == m, iota_f, big), axis=1, keepdims=True)
        cols.append(jf)
        nd = jnp.where(iota_f == jf, neg, nd)
    idx_ref[0] = jnp.concatenate(cols, axis=1).astype(jnp.int32)

    w1 = w_ref[:, :C]
    w21 = w_ref[:, C:] - w1
    pt_ref[0] = lax.dot_general(xs, w1, (((0,), (1,)), ((), ())),
                                preferred_element_type=jnp.float32)
    qt_ref[0] = lax.dot_general(xs, w21, (((0,), (1,)), ((), ())),
                                preferred_element_type=jnp.float32)


_knn_call = pl.pallas_call(
    _knn_body,
    grid=(N // RT,),
    in_specs=[
        pl.BlockSpec((1, C, N), lambda i: (0, 0, 0)),
        pl.BlockSpec((1, C, RT), lambda i: (0, 0, i)),
        pl.BlockSpec((OUT, 2 * C), lambda i: (0, 0)),
    ],
    out_specs=[
        pl.BlockSpec((1, RT, OUT), lambda i: (0, i, 0)),
        pl.BlockSpec((1, RT, OUT), lambda i: (0, i, 0)),
        pl.BlockSpec((1, RT, K), lambda i: (0, i, 0)),
    ],
    out_shape=[
        jax.ShapeDtypeStruct((1, N, OUT), jnp.float32),
        jax.ShapeDtypeStruct((1, N, OUT), jnp.float32),
        jax.ShapeDtypeStruct((1, N, K), jnp.int32),
    ],
)

SEG_PER_W = N // NW


@functools.partial(
    pl.kernel,
    mesh=plsc.VectorSubcoreMesh(core_axis_name="c", subcore_axis_name="s"),
    out_type=[
        jax.ShapeDtypeStruct((N, OUT), jnp.float32),
        jax.ShapeDtypeStruct((NW * 3 * OUT,), jnp.float32),
    ],
    scratch_types=[
        pltpu.VMEM((SEG_PER_W * K,), jnp.int32),
        pltpu.VMEM((SEG_PER_W, OUT), jnp.float32),
        pltpu.VMEM((SEG_PER_W, OUT), jnp.float32),
        pltpu.VMEM((SB * K, OUT), jnp.float32),
        pltpu.VMEM((SB * K, OUT), jnp.float32),
        pltpu.VMEM((3 * OUT,), jnp.float32),
        pltpu.SemaphoreType.DMA,
        pltpu.SemaphoreType.DMA,
    ],
)
def _sc_gather(pt_hbm, idx_hbm, q_hbm, m_hbm, part_hbm,
               idx_v, q_v, m_v, rows0_v, rows1_v, acc_v, sem0, sem1):
    wid = lax.axis_index("s") * NC + lax.axis_index("c")
    base = wid * SEG_PER_W
    nb = SEG_PER_W // SB
    pltpu.sync_copy(idx_hbm.at[pl.ds(base * K, SEG_PER_W * K)], idx_v)
    pltpu.sync_copy(q_hbm.at[pl.ds(base, SEG_PER_W)], q_v)
    zero = jnp.zeros((LANES,), jnp.float32)
    for c in range(3 * OUT // LANES):
        acc_v[pl.ds(c * LANES, LANES)] = zero

    def fire(g, rows_v, sem):
        return pltpu.async_copy(
            pt_hbm.at[idx_v.at[pl.ds(g * SB * K, SB * K)]], rows_v, sem)

    def drain(g, rows_v, sem):
        pltpu.make_async_copy(
            pt_hbm.at[idx_v.at[pl.ds(g * SB * K, SB * K)]], rows_v, sem
        ).wait()

    def process(g, rows_v):
        def chunk_body(c, carry):
            sl = pl.ds(c * LANES, LANES)
            for s in range(SB):
                seg = g * SB + s
                v0 = rows_v[s * K, sl]
                mx = v0
                sm = v0
                sq = v0 * v0
                for j in range(1, K):
                    v = rows_v[s * K + j, sl]
                    mx = jnp.maximum(mx, v)
                    sm = sm + v
                    sq = sq + v * v
                m_v[seg, sl] = mx
                qv = q_v[seg, sl]
                a0 = pl.ds(c * LANES, LANES)
                a1 = pl.ds(OUT + c * LANES, LANES)
                a2 = pl.ds(2 * OUT + c * LANES, LANES)
                acc_v[a0] = acc_v[a0] + sm
                acc_v[a1] = acc_v[a1] + sq
                acc_v[a2] = acc_v[a2] + qv * sm
            return carry

        lax.fori_loop(0, OUT // LANES, chunk_body, 0)

    fire(0, rows0_v, sem0)

    def body(h, carry):
        g0 = 2 * h
        g1 = 2 * h + 1

        @pl.when(g1 < nb)
        def _():
            fire(g1, rows1_v, sem1)
        drain(g0, rows0_v, sem0)
        process(g0, rows0_v)

        @pl.when(g1 < nb)
        def _():
            @pl.when(g1 + 1 < nb)
            def _():
                fire(g1 + 1, rows0_v, sem0)
            drain(g1, rows1_v, sem1)
            process(g1, rows1_v)
        return carry

    lax.fori_loop(0, (nb + 1) // 2, body, 0)
    pltpu.sync_copy(m_v, m_hbm.at[pl.ds(base, SEG_PER_W)])
    pltpu.sync_copy(acc_v, part_hbm.at[pl.ds(wid * 3 * OUT, 3 * OUT)])


def _fin_body(m_ref, q_ref, p_ref, out_ref):
    m = m_ref[...]
    q = q_ref[0]
    p = p_ref[...]
    tsum = jnp.sum(p[:, 0, :], axis=0, keepdims=True)
    psq = jnp.sum(p[:, 1, :], axis=0, keepdims=True)
    cross = jnp.sum(p[:, 2, :], axis=0, keepdims=True)
    qs = jnp.sum(q, axis=0, keepdims=True)
    qs2 = jnp.sum(q * q, axis=0, keepdims=True)
    cnt = jnp.float32(N * K)
    mean = (tsum + K * qs) / cnt
    var = (psq + 2.0 * cross + K * qs2) / cnt - mean * mean
    inv = 1.0 / jnp.sqrt(var + 1e-5)
    z = (m + q - mean) * inv
    out_ref[0] = jnp.where(z >= 0, z, 0.2 * z)


_fin_call = pl.pallas_call(
    _fin_body,
    in_specs=[
        pl.BlockSpec((N, OUT), lambda: (0, 0)),
        pl.BlockSpec((1, N, OUT), lambda: (0, 0, 0)),
        pl.BlockSpec((NW, 3, OUT), lambda: (0, 0, 0)),
    ],
    out_specs=pl.BlockSpec((1, N, OUT), lambda: (0, 0, 0)),
    out_shape=jax.ShapeDtypeStruct((1, N, OUT), jnp.float32),
)


def kernel(cloud, W):
    knn = []
    for b in range(B):
        xb = lax.slice_in_dim(cloud, b, b + 1, axis=0)
        knn.append(_knn_call(xb, xb, W))
    gath = []
    for pt, qt, idx in knn:
        gath.append(_sc_gather(
            pt.reshape(N, OUT), idx.reshape(N * K), qt.reshape(N, OUT)))
    outs = []
    for (pt, qt, idx), (m, parts_flat) in zip(knn, gath):
        outs.append(_fin_call(m, qt, parts_flat.reshape(NW, 3, OUT)))
    out = jnp.concatenate(outs, axis=0)
    return jnp.transpose(out, (0, 2, 1))
